# baseline (speedup 1.0000x reference)
import functools

import jax
import jax.numpy as jnp
from jax import lax
from jax.experimental import pallas as pl
from jax.experimental.pallas import tpu as pltpu
from jax.experimental.pallas import tpu_sc as plsc

NC = 2
NS = 16
NW = NC * NS
CHUNK = 128
TW = 48
PAD = 128 - 2 * TW
D = 64
EPS = 1e-8

BLK = 6400
NSLICE = 5


def _sc_gather(table, zrow, src_idx, dst_idx):
    E = src_idx.shape[0]
    per_w = E // NW
    assert per_w * NW == E and per_w % 8 == 0 and (per_w - CHUNK) % 8 == 0
    n_chunks = (per_w + CHUNK - 1) // CHUNK
    last = per_w - CHUNK

    mesh = plsc.VectorSubcoreMesh(core_axis_name="c", subcore_axis_name="s")

    @functools.partial(
        pl.kernel,
        out_type=jax.ShapeDtypeStruct((E, 2 * D), jnp.float32),
        mesh=mesh,
        scratch_types=[
            pltpu.VMEM((per_w,), jnp.int32),
            pltpu.VMEM((per_w,), jnp.int32),
            pltpu.VMEM((CHUNK, TW), jnp.float32),
            pltpu.VMEM((CHUNK, TW), jnp.float32),
            pltpu.VMEM((CHUNK, PAD), jnp.float32),
            pltpu.SemaphoreType.DMA,
            pltpu.SemaphoreType.DMA,
        ],
        compiler_params=pltpu.CompilerParams(use_tc_tiling_on_sc=False),
    )
    def k(src_hbm, dst_hbm, table_hbm, z_hbm, out_hbm, si_v, di_v, ra, rb,
          zv, sem_a, sem_b):
        wid = lax.axis_index("s") * NC + lax.axis_index("c")
        base = wid * per_w
        pltpu.sync_copy(src_hbm.at[pl.ds(base, per_w)], si_v)
        pltpu.sync_copy(dst_hbm.at[pl.ds(base, per_w)], di_v)
        pltpu.sync_copy(z_hbm, zv)

        def off(kk):
            return jnp.minimum(kk * CHUNK, last)

        def start(idx_v, o, buf, sem):
            pltpu.async_copy(table_hbm.at[idx_v.at[pl.ds(o, CHUNK)]],
                             buf, sem)

        def wait(buf, sem):
            pltpu.make_async_copy(table_hbm.at[si_v.at[pl.ds(0, CHUNK)]],
                                  buf, sem).wait()

        start(si_v, 0, ra, sem_a)

        def body(kk, carry):
            o = off(kk)
            rows = pl.ds(base + o, CHUNK)
            start(di_v, o, rb, sem_b)
            wait(ra, sem_a)
            pltpu.sync_copy(ra, out_hbm.at[rows, pl.ds(0, TW)])
            o1 = off(kk + 1)
            start(si_v, o1, ra, sem_a)
            wait(rb, sem_b)
            pltpu.sync_copy(rb, out_hbm.at[rows, pl.ds(TW, TW)])
            pltpu.sync_copy(zv, out_hbm.at[rows, pl.ds(2 * TW, PAD)])
            return carry

        lax.fori_loop(0, n_chunks, body, 0)
        wait(ra, sem_a)

    return k(src_idx, dst_idx, table, zrow)


def _sqn(x, y, z):
    return x * x + y * y + z * z


def _norm3(x, y, z):
    return jnp.sqrt(_sqn(x, y, z) + EPS)


def _tc_body(gref, hesT, hevT,
             w_all, ws_hevn, m1_b, m1_wv, wh_e,
             m2_ws, m2_vnc, m2_b, m3_ws, m3_vnc, m3_b, scal,
             ln0_g, ln0_b, w_mean,
             f1_ws, f1_b,
             f2_ws, f2_b,
             ln1_g, ln1_b,
             xs_out, xv_out):
    bf16 = jnp.bfloat16

    def dotw(w_ref, x):
        return jax.lax.dot_general(w_ref[...], x,
                                   (((1,), (0,)), ((), ())),
                                   preferred_element_type=jnp.float32)

    def k(i):
        return scal[0, i]

    he_s = hesT[...]
    he_v = hevT[...]
    vex = he_v[0:1]
    vey = he_v[1:2]
    vez = he_v[2:3]

    out = jax.lax.dot_general(w_all[...], gref[...].astype(bf16),
                              (((1,), (1,)), ((), ())),
                              preferred_element_type=jnp.float32)
    whe = wh_e[...]
    vhx = out[32:41] + whe * vex
    vhy = out[48:57] + whe * vey
    vhz = out[64:73] + whe * vez
    vn = _norm3(vhx, vhy, vhz)
    he_vn = jnp.concatenate([he_s.astype(bf16), vn.astype(bf16)], axis=0)
    s = out[0:32] + dotw(ws_hevn, he_vn) + m1_b[...]
    wv = m1_wv[...]
    vox = jnp.sum(vhx * wv, axis=0, keepdims=True)
    voy = jnp.sum(vhy * wv, axis=0, keepdims=True)
    voz = jnp.sum(vhz * wv, axis=0, keepdims=True)
    gate = jax.nn.sigmoid(_norm3(vox, voy, voz))
    vox, voy, voz = vox * gate, voy * gate, voz * gate
    s = jnp.maximum(s, 0.0)

    vhx, vhy, vhz = vox * k(0), voy * k(0), voz * k(0)
    vn1 = _norm3(vhx, vhy, vhz)
    s = dotw(m2_ws, s.astype(bf16)) + m2_vnc[...] * vn1 + m2_b[...]
    vox, voy, voz = vhx * k(1), vhy * k(1), vhz * k(1)
    gate = jax.nn.sigmoid(_norm3(vox, voy, voz))
    vox, voy, voz = vox * gate, voy * gate, voz * gate
    s = jnp.maximum(s, 0.0)

    vhx, vhy, vhz = vox * k(2), voy * k(2), voz * k(2)
    vn1 = _norm3(vhx, vhy, vhz)
    s = dotw(m3_ws, s.astype(bf16)) + m3_vnc[...] * vn1 + m3_b[...]
    vox, voy, voz = vhx * k(3), vhy * k(3), vhz * k(3)

    xs = he_s + s
    xvx, xvy, xvz = vex + vox, vey + voy, vez + voz

    def gvp_ln(xs, xvx, xvy, xvz, g_, b_):
        rn = jax.lax.rsqrt(_sqn(xvx, xvy, xvz) + EPS)
        xvx, xvy, xvz = xvx * rn, xvy * rn, xvz * rn
        xb = xs.astype(bf16)
        mu = dotw(w_mean, xb)
        ex2 = dotw(w_mean, (xs * xs).astype(bf16))
        var = ex2 - mu * mu
        xs = (xs - mu) * jax.lax.rsqrt(var + 1e-5) * g_[...] + b_[...]
        return xs, xvx, xvy, xvz

    xs, xvx, xvy, xvz = gvp_ln(xs, xvx, xvy, xvz, ln0_g, ln0_b)

    h0x, h0y, h0z = xvx * k(4), xvy * k(4), xvz * k(4)
    h1x, h1y, h1z = xvx * k(5), xvy * k(5), xvz * k(5)
    vn0 = _norm3(h0x, h0y, h0z)
    vn1 = _norm3(h1x, h1y, h1z)
    x34 = jnp.concatenate([xs.astype(bf16), vn0.astype(bf16),
                           vn1.astype(bf16)], axis=0)
    ds = dotw(f1_ws, x34).astype(bf16) + f1_b[...]
    o0x = k(6) * h0x + k(7) * h1x
    o0y = k(6) * h0y + k(7) * h1y
    o0z = k(6) * h0z + k(7) * h1z
    o1x = k(8) * h0x + k(9) * h1x
    o1y = k(8) * h0y + k(9) * h1y
    o1z = k(8) * h0z + k(9) * h1z
    g0 = jax.nn.sigmoid(_norm3(o0x, o0y, o0z))
    g1 = jax.nn.sigmoid(_norm3(o1x, o1y, o1z))
    o0x, o0y, o0z = o0x * g0, o0y * g0, o0z * g0
    o1x, o1y, o1z = o1x * g1, o1y * g1, o1z * g1
    ds = jnp.maximum(ds, jnp.zeros((), bf16))

    h0x = k(10) * o0x + k(11) * o1x
    h0y = k(10) * o0y + k(11) * o1y
    h0z = k(10) * o0z + k(11) * o1z
    h1x = k(12) * o0x + k(13) * o1x
    h1y = k(12) * o0y + k(13) * o1y
    h1z = k(12) * o0z + k(13) * o1z
    vn0 = _norm3(h0x, h0y, h0z)
    vn1 = _norm3(h1x, h1y, h1z)
    x130 = jnp.concatenate([ds, vn0.astype(bf16), vn1.astype(bf16)],
                           axis=0)
    ds = dotw(f2_ws, x130) + f2_b[...]
    vox = k(14) * h0x + k(15) * h1x
    voy = k(14) * h0y + k(15) * h1y
    voz = k(14) * h0z + k(15) * h1z

    xs = xs + ds
    xvx, xvy, xvz = xvx + vox, xvy + voy, xvz + voz
    xs, xvx, xvy, xvz = gvp_ln(xs, xvx, xvy, xvz, ln1_g, ln1_b)

    xs_out[...] = xs
    xv_out[...] = jnp.concatenate([xvx, xvy, xvz], axis=0)


def _build_weights(p):
    f32 = jnp.float32
    bf16 = jnp.bfloat16
    m1_ws = p['m1_ws_w']
    m1_wh = p['m1_wh']
    w_all = jnp.zeros((80, 128), f32)
    w_all = w_all.at[0:32, 0:32].set(m1_ws[:, 0:32])
    w_all = w_all.at[0:32, 48:80].set(m1_ws[:, 64:96])
    for comp, (r0, c_i, c_j) in enumerate([(32, 32, 80), (48, 36, 84),
                                           (64, 40, 88)]):
        w_all = w_all.at[r0:r0 + 9, c_i:c_i + 4].set(m1_wh[:, 0:4])
        w_all = w_all.at[r0:r0 + 9, c_j:c_j + 4].set(m1_wh[:, 5:9])
    scal = jnp.stack([
        p['m2_wh'][0, 0], p['m2_wv'][0, 0],
        p['m3_wh'][0, 0], p['m3_wv'][0, 0],
        p['f1_wh'][0, 0], p['f1_wh'][1, 0],
        p['f1_wv'][0, 0], p['f1_wv'][0, 1],
        p['f1_wv'][1, 0], p['f1_wv'][1, 1],
        p['f2_wh'][0, 0], p['f2_wh'][0, 1],
        p['f2_wh'][1, 0], p['f2_wh'][1, 1],
        p['f2_wv'][0, 0], p['f2_wv'][0, 1],
    ]).reshape(1, 16)
    return [
        w_all.astype(bf16),
        jnp.concatenate([m1_ws[:, 32:64], m1_ws[:, 96:105]],
                        axis=1).astype(bf16),
        p['m1_ws_b'].reshape(32, 1),
        p['m1_wv'].reshape(9, 1),
        m1_wh[:, 4].reshape(9, 1),
        p['m2_ws_w'][:, 0:32].astype(bf16),
        p['m2_ws_w'][:, 32:33],
        p['m2_ws_b'].reshape(32, 1),
        p['m3_ws_w'][:, 0:32].astype(bf16),
        p['m3_ws_w'][:, 32:33],
        p['m3_ws_b'].reshape(32, 1),
        scal,
        p['ln0_g'].reshape(32, 1), p['ln0_b'].reshape(32, 1),
        jnp.full((1, 32), 1.0 / 32.0, bf16),
        p['f1_ws_w'].astype(bf16),
        p['f1_ws_b'].reshape(128, 1).astype(bf16),
        p['f2_ws_w'].astype(bf16),
        p['f2_ws_b'].reshape(32, 1),
        p['ln1_g'].reshape(32, 1), p['ln1_b'].reshape(32, 1),
    ]


def kernel(h_V_s, h_V_v, edge_index, h_E_s, h_E_v, params):
    N, si = h_V_s.shape
    E = edge_index.shape[1]
    f32 = jnp.float32

    table = jnp.concatenate(
        [h_V_s, h_V_v[:, :, 0], h_V_v[:, :, 1], h_V_v[:, :, 2],
         jnp.zeros((N, TW - si - 12), f32)], axis=1)
    zrow = jnp.zeros((CHUNK, PAD), f32)

    hesT = h_E_s.T
    hevT = h_E_v.reshape(E, 3).T
    weights = _build_weights(params)

    sizes = ([32000, 224000, 224000, 224000, 96000]
             if E == 800000 else [E])
    assert sum(sizes) == E

    def wspec(a):
        return pl.BlockSpec(a.shape, lambda i: tuple(0 for _ in a.shape))

    xs_parts = []
    xv_parts = []
    lo = 0
    for Es in sizes:
        nb = Es // BLK
        ob = lo // BLK
        assert nb * BLK == Es and ob * BLK == lo
        g_s = _sc_gather(table, zrow,
                         lax.slice(edge_index[0], (lo,), (lo + Es,)),
                         lax.slice(edge_index[1], (lo,), (lo + Es,)))
        lo += Es

        def espec(c, ob=ob):
            return pl.BlockSpec((c, BLK), lambda i, ob=ob: (0, i + ob))

        in_specs = ([pl.BlockSpec((BLK, 2 * D), lambda i: (i, 0)),
                     espec(si), espec(3)]
                    + [wspec(w) for w in weights])

        xs_fm, xv_fm = pl.pallas_call(
            _tc_body,
            grid=(nb,),
            in_specs=in_specs,
            out_specs=[pl.BlockSpec((si, BLK), lambda i: (0, i)),
                       pl.BlockSpec((3, BLK), lambda i: (0, i))],
            out_shape=[jax.ShapeDtypeStruct((si, Es), f32),
                       jax.ShapeDtypeStruct((3, Es), f32)],
        )(g_s, hesT, hevT, *weights)
        xs_parts.append(xs_fm.T)
        xv_parts.append(xv_fm.T)

    xs = jnp.concatenate(xs_parts, axis=0)
    xv = jnp.concatenate(xv_parts, axis=0)
    return xs, xv.reshape(E, 1, 3)


# speedup vs baseline: 1.0104x; 1.0104x over previous
import functools

import jax
import jax.numpy as jnp
from jax import lax
from jax.experimental import pallas as pl
from jax.experimental.pallas import tpu as pltpu
from jax.experimental.pallas import tpu_sc as plsc

NC = 2
NS = 16
NW = NC * NS
CHUNK = 128
TW = 48
D = 64
EPS = 1e-8

BLK = 6400
NSLICE = 5


def _sc_gather(table, zrow, src_idx, dst_idx):
    E = src_idx.shape[0]
    per_w = E // NW
    assert per_w * NW == E and per_w % 8 == 0 and (per_w - CHUNK) % 8 == 0
    n_chunks = (per_w + CHUNK - 1) // CHUNK
    last = per_w - CHUNK

    mesh = plsc.VectorSubcoreMesh(core_axis_name="c", subcore_axis_name="s")

    @functools.partial(
        pl.kernel,
        out_type=jax.ShapeDtypeStruct((E, 2 * D), jnp.float32),
        mesh=mesh,
        scratch_types=[
            pltpu.VMEM((per_w,), jnp.int32),
            pltpu.VMEM((per_w,), jnp.int32),
            pltpu.VMEM((CHUNK, TW), jnp.float32),
            pltpu.VMEM((CHUNK, TW), jnp.float32),
            pltpu.VMEM((CHUNK, D - TW), jnp.float32),
            pltpu.SemaphoreType.DMA,
            pltpu.SemaphoreType.DMA,
        ],
        compiler_params=pltpu.CompilerParams(use_tc_tiling_on_sc=False),
    )
    def k(src_hbm, dst_hbm, table_hbm, z_hbm, out_hbm, si_v, di_v, ra, rb,
          zv, sem_a, sem_b):
        wid = lax.axis_index("s") * NC + lax.axis_index("c")
        base = wid * per_w
        pltpu.sync_copy(src_hbm.at[pl.ds(base, per_w)], si_v)
        pltpu.sync_copy(dst_hbm.at[pl.ds(base, per_w)], di_v)
        pltpu.sync_copy(z_hbm, zv)

        def off(kk):
            return jnp.minimum(kk * CHUNK, last)

        def start(idx_v, o, buf, sem):
            pltpu.async_copy(table_hbm.at[idx_v.at[pl.ds(o, CHUNK)]],
                             buf, sem)

        def wait(buf, sem):
            pltpu.make_async_copy(table_hbm.at[si_v.at[pl.ds(0, CHUNK)]],
                                  buf, sem).wait()

        start(si_v, 0, ra, sem_a)

        def body(kk, carry):
            o = off(kk)
            rows = pl.ds(base + o, CHUNK)
            start(di_v, o, rb, sem_b)
            wait(ra, sem_a)
            pltpu.sync_copy(ra, out_hbm.at[rows, pl.ds(0, TW)])
            pltpu.sync_copy(zv, out_hbm.at[rows, pl.ds(TW, D - TW)])
            o1 = off(kk + 1)
            start(si_v, o1, ra, sem_a)
            wait(rb, sem_b)
            pltpu.sync_copy(rb, out_hbm.at[rows, pl.ds(D, TW)])
            pltpu.sync_copy(zv, out_hbm.at[rows, pl.ds(D + TW, D - TW)])
            return carry

        lax.fori_loop(0, n_chunks, body, 0)
        wait(ra, sem_a)

    return k(src_idx, dst_idx, table, zrow)


def _sqn(x, y, z):
    return x * x + y * y + z * z


def _norm3(x, y, z):
    return jnp.sqrt(_sqn(x, y, z) + EPS)


def _tc_body(gref, hesT, hevT,
             w_all, ws_hevn, m1_b, m1_wv, wh_e,
             m2_ws, m2_vnc, m2_b, m3_ws, m3_vnc, m3_b, scal,
             ln0_g, ln0_b, w_mean,
             f1_ws, f1_b,
             f2_ws, f2_b,
             ln1_g, ln1_b,
             xs_out, xv_out):
    bf16 = jnp.bfloat16

    def dotw(w_ref, x):
        return jax.lax.dot_general(w_ref[...], x,
                                   (((1,), (0,)), ((), ())),
                                   preferred_element_type=jnp.float32)

    def k(i):
        return scal[0, i]

    he_s = hesT[...]
    he_v = hevT[...]
    vex = he_v[0:1]
    vey = he_v[1:2]
    vez = he_v[2:3]

    out = jax.lax.dot_general(w_all[...], gref[...].astype(bf16),
                              (((1,), (1,)), ((), ())),
                              preferred_element_type=jnp.float32)
    whe = wh_e[...]
    vhx = out[32:41] + whe * vex
    vhy = out[48:57] + whe * vey
    vhz = out[64:73] + whe * vez
    vn = _norm3(vhx, vhy, vhz)
    he_vn = jnp.concatenate([he_s.astype(bf16), vn.astype(bf16)], axis=0)
    s = out[0:32] + dotw(ws_hevn, he_vn) + m1_b[...]
    wv = m1_wv[...]
    vox = jnp.sum(vhx * wv, axis=0, keepdims=True)
    voy = jnp.sum(vhy * wv, axis=0, keepdims=True)
    voz = jnp.sum(vhz * wv, axis=0, keepdims=True)
    gate = jax.nn.sigmoid(_norm3(vox, voy, voz))
    vox, voy, voz = vox * gate, voy * gate, voz * gate
    s = jnp.maximum(s, 0.0)

    vhx, vhy, vhz = vox * k(0), voy * k(0), voz * k(0)
    vn1 = _norm3(vhx, vhy, vhz)
    s = dotw(m2_ws, s.astype(bf16)) + m2_vnc[...] * vn1 + m2_b[...]
    vox, voy, voz = vhx * k(1), vhy * k(1), vhz * k(1)
    gate = jax.nn.sigmoid(_norm3(vox, voy, voz))
    vox, voy, voz = vox * gate, voy * gate, voz * gate
    s = jnp.maximum(s, 0.0)

    vhx, vhy, vhz = vox * k(2), voy * k(2), voz * k(2)
    vn1 = _norm3(vhx, vhy, vhz)
    s = dotw(m3_ws, s.astype(bf16)) + m3_vnc[...] * vn1 + m3_b[...]
    vox, voy, voz = vhx * k(3), vhy * k(3), vhz * k(3)

    xs = he_s + s
    xvx, xvy, xvz = vex + vox, vey + voy, vez + voz

    def gvp_ln(xs, xvx, xvy, xvz, g_, b_):
        rn = jax.lax.rsqrt(_sqn(xvx, xvy, xvz) + EPS)
        xvx, xvy, xvz = xvx * rn, xvy * rn, xvz * rn
        xb = xs.astype(bf16)
        mu = dotw(w_mean, xb)
        ex2 = dotw(w_mean, (xs * xs).astype(bf16))
        var = ex2 - mu * mu
        xs = (xs - mu) * jax.lax.rsqrt(var + 1e-5) * g_[...] + b_[...]
        return xs, xvx, xvy, xvz

    xs, xvx, xvy, xvz = gvp_ln(xs, xvx, xvy, xvz, ln0_g, ln0_b)

    h0x, h0y, h0z = xvx * k(4), xvy * k(4), xvz * k(4)
    h1x, h1y, h1z = xvx * k(5), xvy * k(5), xvz * k(5)
    vn0 = _norm3(h0x, h0y, h0z)
    vn1 = _norm3(h1x, h1y, h1z)
    x34 = jnp.concatenate([xs.astype(bf16), vn0.astype(bf16),
                           vn1.astype(bf16)], axis=0)
    ds = dotw(f1_ws, x34).astype(bf16) + f1_b[...]
    o0x = k(6) * h0x + k(7) * h1x
    o0y = k(6) * h0y + k(7) * h1y
    o0z = k(6) * h0z + k(7) * h1z
    o1x = k(8) * h0x + k(9) * h1x
    o1y = k(8) * h0y + k(9) * h1y
    o1z = k(8) * h0z + k(9) * h1z
    g0 = jax.nn.sigmoid(_norm3(o0x, o0y, o0z))
    g1 = jax.nn.sigmoid(_norm3(o1x, o1y, o1z))
    o0x, o0y, o0z = o0x * g0, o0y * g0, o0z * g0
    o1x, o1y, o1z = o1x * g1, o1y * g1, o1z * g1
    ds = jnp.maximum(ds, jnp.zeros((), bf16))

    h0x = k(10) * o0x + k(11) * o1x
    h0y = k(10) * o0y + k(11) * o1y
    h0z = k(10) * o0z + k(11) * o1z
    h1x = k(12) * o0x + k(13) * o1x
    h1y = k(12) * o0y + k(13) * o1y
    h1z = k(12) * o0z + k(13) * o1z
    vn0 = _norm3(h0x, h0y, h0z)
    vn1 = _norm3(h1x, h1y, h1z)
    x130 = jnp.concatenate([ds, vn0.astype(bf16), vn1.astype(bf16)],
                           axis=0)
    ds = dotw(f2_ws, x130) + f2_b[...]
    vox = k(14) * h0x + k(15) * h1x
    voy = k(14) * h0y + k(15) * h1y
    voz = k(14) * h0z + k(15) * h1z

    xs = xs + ds
    xvx, xvy, xvz = xvx + vox, xvy + voy, xvz + voz
    xs, xvx, xvy, xvz = gvp_ln(xs, xvx, xvy, xvz, ln1_g, ln1_b)

    xs_out[...] = xs
    xv_out[...] = jnp.concatenate([xvx, xvy, xvz], axis=0)


def _build_weights(p):
    f32 = jnp.float32
    bf16 = jnp.bfloat16
    m1_ws = p['m1_ws_w']
    m1_wh = p['m1_wh']
    w_all = jnp.zeros((80, 128), f32)
    w_all = w_all.at[0:32, 0:32].set(m1_ws[:, 0:32])
    w_all = w_all.at[0:32, 64:96].set(m1_ws[:, 64:96])
    for comp, (r0, c_i, c_j) in enumerate([(32, 32, 96), (48, 36, 100),
                                           (64, 40, 104)]):
        w_all = w_all.at[r0:r0 + 9, c_i:c_i + 4].set(m1_wh[:, 0:4])
        w_all = w_all.at[r0:r0 + 9, c_j:c_j + 4].set(m1_wh[:, 5:9])
    scal = jnp.stack([
        p['m2_wh'][0, 0], p['m2_wv'][0, 0],
        p['m3_wh'][0, 0], p['m3_wv'][0, 0],
        p['f1_wh'][0, 0], p['f1_wh'][1, 0],
        p['f1_wv'][0, 0], p['f1_wv'][0, 1],
        p['f1_wv'][1, 0], p['f1_wv'][1, 1],
        p['f2_wh'][0, 0], p['f2_wh'][0, 1],
        p['f2_wh'][1, 0], p['f2_wh'][1, 1],
        p['f2_wv'][0, 0], p['f2_wv'][0, 1],
    ]).reshape(1, 16)
    return [
        w_all.astype(bf16),
        jnp.concatenate([m1_ws[:, 32:64], m1_ws[:, 96:105]],
                        axis=1).astype(bf16),
        p['m1_ws_b'].reshape(32, 1),
        p['m1_wv'].reshape(9, 1),
        m1_wh[:, 4].reshape(9, 1),
        p['m2_ws_w'][:, 0:32].astype(bf16),
        p['m2_ws_w'][:, 32:33],
        p['m2_ws_b'].reshape(32, 1),
        p['m3_ws_w'][:, 0:32].astype(bf16),
        p['m3_ws_w'][:, 32:33],
        p['m3_ws_b'].reshape(32, 1),
        scal,
        p['ln0_g'].reshape(32, 1), p['ln0_b'].reshape(32, 1),
        jnp.full((1, 32), 1.0 / 32.0, bf16),
        p['f1_ws_w'].astype(bf16),
        p['f1_ws_b'].reshape(128, 1).astype(bf16),
        p['f2_ws_w'].astype(bf16),
        p['f2_ws_b'].reshape(32, 1),
        p['ln1_g'].reshape(32, 1), p['ln1_b'].reshape(32, 1),
    ]


def kernel(h_V_s, h_V_v, edge_index, h_E_s, h_E_v, params):
    N, si = h_V_s.shape
    E = edge_index.shape[1]
    f32 = jnp.float32

    table = jnp.concatenate(
        [h_V_s, h_V_v[:, :, 0], h_V_v[:, :, 1], h_V_v[:, :, 2],
         jnp.zeros((N, TW - si - 12), f32)], axis=1)
    zrow = jnp.zeros((CHUNK, D - TW), f32)

    hesT = h_E_s.T
    hevT = h_E_v.reshape(E, 3).T
    weights = _build_weights(params)

    Es = E // NSLICE
    nb = Es // BLK
    assert Es * NSLICE == E and nb * BLK == Es

    def wspec(a):
        return pl.BlockSpec(a.shape, lambda i: tuple(0 for _ in a.shape))

    xs_parts = []
    xv_parts = []
    for sl in range(NSLICE):
        lo = sl * Es
        g_s = _sc_gather(table, zrow,
                         lax.slice(edge_index[0], (lo,), (lo + Es,)),
                         lax.slice(edge_index[1], (lo,), (lo + Es,)))

        def espec(c, sl=sl):
            return pl.BlockSpec((c, BLK), lambda i, sl=sl: (0, i + sl * nb))

        in_specs = ([pl.BlockSpec((BLK, 2 * D), lambda i: (i, 0)),
                     espec(si), espec(3)]
                    + [wspec(w) for w in weights])

        xs_fm, xv_fm = pl.pallas_call(
            _tc_body,
            grid=(nb,),
            in_specs=in_specs,
            out_specs=[pl.BlockSpec((si, BLK), lambda i: (0, i)),
                       pl.BlockSpec((3, BLK), lambda i: (0, i))],
            out_shape=[jax.ShapeDtypeStruct((si, Es), f32),
                       jax.ShapeDtypeStruct((3, Es), f32)],
        )(g_s, hesT, hevT, *weights)
        xs_parts.append(xs_fm.T)
        xv_parts.append(xv_fm.T)

    xs = jnp.concatenate(xs_parts, axis=0)
    xv = jnp.concatenate(xv_parts, axis=0)
    return xs, xv.reshape(E, 1, 3)
